# baseline (speedup 1.0000x reference)
import functools

import jax
import jax.numpy as jnp
from jax import lax
from jax.experimental import pallas as pl
from jax.experimental.pallas import tpu as pltpu
from jax.experimental.pallas import tpu_sc as plsc

NC = 2
NS = 16
L = 16
NW = NC * NS
BLK = 128
CHUNK = 1024
BN = 1024


def _sc_mesh():
    return plsc.VectorSubcoreMesh(core_axis_name="c", subcore_axis_name="s",
                                  num_cores=NC)


_SC_PARAMS = pltpu.CompilerParams(needs_layout_passes=False,
                                  use_tc_tiling_on_sc=False)


def _make_deg_kernel(epad, npad):
    ec = 512
    nb = ec // BLK
    eblocks = epad // BLK
    bpt = eblocks // NW
    nchunks = bpt // nb
    rpt = npad // NS
    NBUF = 3

    @functools.partial(
        pl.kernel,
        out_type=jax.ShapeDtypeStruct((NC, npad, L), jnp.float32),
        mesh=_sc_mesh(),
        scratch_types=(
            [pltpu.VMEM((nb, BLK), jnp.int32) for _ in range(NBUF)]
            + [pltpu.VMEM((ec,), jnp.float32) for _ in range(NBUF)]
            + [pltpu.VMEM((ec, L), jnp.float32) for _ in range(NBUF)]
            + [pltpu.VMEM_SHARED((npad, L), jnp.float32)]
            + [pltpu.SemaphoreType.DMA for _ in range(NBUF)]
        ),
        compiler_params=_SC_PARAMS,
    )
    def deg_kernel(dst_hbm, ewf_hbm, out_hbm, *scr):
        didx = scr[0:NBUF]
        ew = scr[NBUF:2 * NBUF]
        rows = scr[2 * NBUF:3 * NBUF]
        acc_sh = scr[3 * NBUF]
        ssem = scr[3 * NBUF + 1:]

        core = lax.axis_index("c")
        sub = lax.axis_index("s")
        wid = core * NS + sub
        r0 = sub * rpt

        zero16 = jnp.zeros((L,), jnp.float32)
        for b in range(NBUF):
            @pl.loop(0, ec)
            def _(i):
                rows[b][i, :] = zero16

        done = 0
        while done < rpt:
            n = min(ec, rpt - done)
            pltpu.sync_copy(rows[0].at[pl.ds(0, n)],
                            acc_sh.at[pl.ds(r0 + done, n)])
            done += n
        plsc.subcore_barrier()

        col0 = jnp.zeros((L,), jnp.int32)
        sd = {}
        for c in range(nchunks):
            b = c % NBUF
            if c - NBUF >= 0:
                for d in sd[c - NBUF]:
                    d.wait()
            blk0 = wid * bpt + c * nb
            pltpu.sync_copy(dst_hbm.at[pl.ds(blk0, nb)], didx[b])
            pltpu.sync_copy(ewf_hbm.at[pl.ds(blk0 * BLK, ec)], ew[b])

            rows_b, ew_b = rows[b], ew[b]

            @plsc.parallel_loop(0, ec, step=L, unroll=2)
            def _(e0):
                eidx = e0 + lax.iota(jnp.int32, L)
                plsc.store_scatter(rows_b, [eidx, col0], ew_b[pl.ds(e0, L)])

            sd[c] = [
                pltpu.async_copy(rows_b.at[pl.ds(j * BLK, BLK)],
                                 acc_sh.at[didx[b].at[j]], ssem[b], add=True)
                for j in range(nb)
            ]

        for c in range(max(0, nchunks - NBUF), nchunks):
            for d in sd[c]:
                d.wait()

        plsc.subcore_barrier()
        pltpu.sync_copy(acc_sh.at[pl.ds(r0, rpt)],
                        out_hbm.at[core, pl.ds(r0, rpt)])

    return deg_kernel


def _make_prop_kernel(epad, npad, D):
    ec = 512
    nb = ec // BLK
    eblocks = epad // BLK
    bpt = eblocks // NW
    nchunks = bpt // nb
    rpt = npad // NS
    NBUF = 3

    @functools.partial(
        pl.kernel,
        out_type=jax.ShapeDtypeStruct((NC, npad, D), jnp.float32),
        mesh=_sc_mesh(),
        scratch_types=(
            [pltpu.VMEM((nb, BLK), jnp.int32) for _ in range(NBUF)]
            + [pltpu.VMEM((nb, BLK), jnp.int32) for _ in range(NBUF)]
            + [pltpu.VMEM((ec,), jnp.float32) for _ in range(NBUF)]
            + [pltpu.VMEM((ec, D), jnp.float32) for _ in range(NBUF)]
            + [pltpu.VMEM_SHARED((npad, D), jnp.float32)]
            + [pltpu.SemaphoreType.DMA for _ in range(2 * NBUF)]
        ),
        compiler_params=_SC_PARAMS,
    )
    def prop_kernel(src_hbm, dst_hbm, ewf_hbm, g_hbm, out_hbm, *scr):
        sidx = scr[0:NBUF]
        didx = scr[NBUF:2 * NBUF]
        ew = scr[2 * NBUF:3 * NBUF]
        rows = scr[3 * NBUF:4 * NBUF]
        acc_sh = scr[4 * NBUF]
        gsem = scr[4 * NBUF + 1:4 * NBUF + 1 + NBUF]
        ssem = scr[4 * NBUF + 1 + NBUF:]

        core = lax.axis_index("c")
        sub = lax.axis_index("s")
        wid = core * NS + sub
        r0 = sub * rpt

        zero16 = jnp.zeros((L,), jnp.float32)

        @pl.loop(0, min(rpt, ec))
        def _(i):
            for j0 in range(0, D, L):
                rows[0][i, pl.ds(j0, L)] = zero16

        @pl.when(core == 0)
        def _():
            pltpu.sync_copy(g_hbm.at[pl.ds(r0, rpt)], acc_sh.at[pl.ds(r0, rpt)])

        @pl.when(core != 0)
        def _():
            done = 0
            while done < rpt:
                n = min(ec, rpt - done)
                pltpu.sync_copy(rows[0].at[pl.ds(0, n)],
                                acc_sh.at[pl.ds(r0 + done, n)])
                done += n

        plsc.subcore_barrier()

        def load_idx(c):
            b = c % NBUF
            blk0 = wid * bpt + c * nb
            pltpu.sync_copy(src_hbm.at[pl.ds(blk0, nb)], sidx[b])
            pltpu.sync_copy(dst_hbm.at[pl.ds(blk0, nb)], didx[b])
            pltpu.sync_copy(ewf_hbm.at[pl.ds(blk0 * BLK, ec)], ew[b])

        def issue_gathers(c):
            b = c % NBUF
            return [
                pltpu.async_copy(g_hbm.at[sidx[b].at[j]],
                                 rows[b].at[pl.ds(j * BLK, BLK)], gsem[b])
                for j in range(nb)
            ]

        def issue_scatters(c):
            b = c % NBUF
            return [
                pltpu.async_copy(rows[b].at[pl.ds(j * BLK, BLK)],
                                 acc_sh.at[didx[b].at[j]], ssem[b], add=True)
                for j in range(nb)
            ]

        lane_ids = [jnp.full((L,), l, jnp.int32) for l in range(L)]

        def scale(c):
            b = c % NBUF
            rows_b = rows[b]
            ew_b = ew[b]

            @plsc.parallel_loop(0, ec, step=L, unroll=2)
            def _(e0):
                wt16 = ew_b[pl.ds(e0, L)]
                for l in range(L):
                    wb = lax.gather(
                        wt16, lane_ids[l][:, None],
                        lax.GatherDimensionNumbers(
                            offset_dims=(), collapsed_slice_dims=(0,),
                            start_index_map=(0,)),
                        (1,), mode=lax.GatherScatterMode.PROMISE_IN_BOUNDS)
                    for j0 in range(0, D, L):
                        rows_b[e0 + l, pl.ds(j0, L)] = (
                            rows_b[e0 + l, pl.ds(j0, L)] * wb)

        gd = {}
        sd = {}
        load_idx(0)
        gd[0] = issue_gathers(0)
        for c in range(nchunks):
            if c + 1 < nchunks:
                if c - 2 >= 0:
                    for d in sd[c - 2]:
                        d.wait()
                load_idx(c + 1)
                gd[c + 1] = issue_gathers(c + 1)
            for d in gd[c]:
                d.wait()
            scale(c)
            sd[c] = issue_scatters(c)
        for c in range(max(0, nchunks - 3), nchunks):
            for d in sd[c]:
                d.wait()

        plsc.subcore_barrier()
        pltpu.sync_copy(acc_sh.at[pl.ds(r0, rpt)],
                        out_hbm.at[core, pl.ds(r0, rpt)])

    return prop_kernel


def _dis_from_deg(degp):
    deg = 1.0 + degp[0, :, 0] + degp[1, :, 0]
    return lax.rsqrt(deg)


def _tc1_body(degp_ref, x_ref, w1_ref, g_ref):
    dis = _dis_from_deg(degp_ref[...])
    h = jnp.dot(x_ref[...], w1_ref[...], preferred_element_type=jnp.float32)
    g_ref[...] = dis[:, None] * h


def _tc2_body(degp_ref, s1_ref, b1_ref, w2_ref, g2_ref):
    dis = _dis_from_deg(degp_ref[...])
    s = s1_ref[...]
    conv1 = dis[:, None] * (s[0] + s[1]) + b1_ref[...]
    o1 = jnp.maximum(conv1, 0.0)
    h2 = jnp.dot(o1, w2_ref[...], preferred_element_type=jnp.float32)
    g2_ref[...] = dis[:, None] * h2


def _tc3_body(C, degp_ref, s2_ref, b2_ref, out_ref):
    dis = _dis_from_deg(degp_ref[...])
    s = s2_ref[...]
    conv2 = dis[:, None] * (s[0] + s[1])[:, :C] + b2_ref[...]
    m = jnp.max(conv2, axis=1, keepdims=True)
    lse = m + jnp.log(jnp.sum(jnp.exp(conv2 - m), axis=1, keepdims=True))
    out_ref[...] = conv2 - lse


def _full2d(shape):
    return pl.BlockSpec(shape, lambda i: (0, 0))


def kernel(x, edge_index, edge_weight, W1, b1, W2, b2):
    N, F = x.shape
    HID = W1.shape[1]
    C = W2.shape[1]
    E = edge_weight.shape[0]

    npad = ((N + BN - 1) // BN) * BN
    epad = ((E + NW * CHUNK - 1) // (NW * CHUNK)) * (NW * CHUNK)
    D2 = ((C + L - 1) // L) * L

    spread = (jnp.arange(epad - E, dtype=jnp.int32) * 37) % N
    src2d = jnp.concatenate([edge_index[0], spread]).reshape(epad // BLK, BLK)
    dst2d = jnp.concatenate([edge_index[1], spread]).reshape(epad // BLK, BLK)
    ewf = jnp.pad(edge_weight, (0, epad - E))
    xp = jnp.pad(x, ((0, npad - N), (0, 0)))
    W2p = jnp.pad(W2, ((0, 0), (0, D2 - C)))
    b1r = b1.reshape(1, HID)
    b2r = b2.reshape(1, C)

    degp = _make_deg_kernel(epad, npad)(dst2d, ewf)

    grid = (npad // BN,)
    degp_spec = pl.BlockSpec((NC, BN, L), lambda i: (0, i, 0))

    g1 = pl.pallas_call(
        _tc1_body,
        grid=grid,
        in_specs=[degp_spec,
                  pl.BlockSpec((BN, F), lambda i: (i, 0)),
                  _full2d((F, HID))],
        out_specs=pl.BlockSpec((BN, HID), lambda i: (i, 0)),
        out_shape=jax.ShapeDtypeStruct((npad, HID), jnp.float32),
    )(degp, xp, W1)

    s1 = _make_prop_kernel(epad, npad, HID)(src2d, dst2d, ewf, g1)

    g2 = pl.pallas_call(
        _tc2_body,
        grid=grid,
        in_specs=[degp_spec,
                  pl.BlockSpec((NC, BN, HID), lambda i: (0, i, 0)),
                  _full2d((1, HID)),
                  _full2d((HID, D2))],
        out_specs=pl.BlockSpec((BN, D2), lambda i: (i, 0)),
        out_shape=jax.ShapeDtypeStruct((npad, D2), jnp.float32),
    )(degp, s1, b1r, W2p)

    s2 = _make_prop_kernel(epad, npad, D2)(src2d, dst2d, ewf, g2)

    out = pl.pallas_call(
        functools.partial(_tc3_body, C),
        grid=grid,
        in_specs=[degp_spec,
                  pl.BlockSpec((NC, BN, D2), lambda i: (0, i, 0)),
                  _full2d((1, C))],
        out_specs=pl.BlockSpec((BN, C), lambda i: (i, 0)),
        out_shape=jax.ShapeDtypeStruct((npad, C), jnp.float32),
    )(degp, s2, b2r)

    return out[:N]


# speedup vs baseline: 1.1448x; 1.1448x over previous
import functools

import jax
import jax.numpy as jnp
from jax import lax
from jax.experimental import pallas as pl
from jax.experimental.pallas import tpu as pltpu
from jax.experimental.pallas import tpu_sc as plsc

NC = 2
NS = 16
L = 16
NW = NC * NS
BLK = 128
CHUNK = 1024
BN = 1024


def _sc_mesh():
    return plsc.VectorSubcoreMesh(core_axis_name="c", subcore_axis_name="s",
                                  num_cores=NC)


_SC_PARAMS = pltpu.CompilerParams(needs_layout_passes=False,
                                  use_tc_tiling_on_sc=False)


def _make_deg_kernel(epad, npad):
    ept = epad // NW
    rpt = npad // NS

    @functools.partial(
        pl.kernel,
        out_type=jax.ShapeDtypeStruct((NC, npad), jnp.float32),
        mesh=_sc_mesh(),
        scratch_types=[
            pltpu.VMEM((npad,), jnp.float32),
            pltpu.VMEM((ept,), jnp.int32),
            pltpu.VMEM((ept,), jnp.float32),
            pltpu.VMEM((NS, rpt), jnp.float32),
            pltpu.VMEM_SHARED((NS, npad), jnp.float32),
            pltpu.SemaphoreType.DMA,
        ],
        compiler_params=_SC_PARAMS,
    )
    def deg_kernel(dstf_hbm, ewf_hbm, out_hbm, hist_v, didx_v, ew_v,
                   red_v, slab_sh, sem):
        core = lax.axis_index("c")
        sub = lax.axis_index("s")
        wid = core * NS + sub
        r0 = sub * rpt

        d0 = pltpu.async_copy(dstf_hbm.at[pl.ds(wid * ept, ept)], didx_v, sem)
        d1 = pltpu.async_copy(ewf_hbm.at[pl.ds(wid * ept, ept)], ew_v, sem)

        zero16 = jnp.zeros((L,), jnp.float32)

        @pl.loop(0, npad, step=L)
        def _(i):
            hist_v[pl.ds(i, L)] = zero16

        d0.wait()
        d1.wait()

        @plsc.parallel_loop(0, ept, step=L, unroll=2)
        def _(e0):
            plsc.addupdate_scatter(hist_v, [didx_v[pl.ds(e0, L)]],
                                   ew_v[pl.ds(e0, L)])

        pltpu.sync_copy(hist_v, slab_sh.at[sub])
        plsc.subcore_barrier()
        pltpu.sync_copy(slab_sh.at[:, pl.ds(r0, rpt)], red_v)

        @pl.loop(0, rpt, step=L)
        def _(j):
            s = red_v[0, pl.ds(j, L)]
            for t in range(1, NS):
                s = s + red_v[t, pl.ds(j, L)]
            hist_v[pl.ds(j, L)] = s

        pltpu.sync_copy(hist_v.at[pl.ds(0, rpt)],
                        out_hbm.at[core, pl.ds(r0, rpt)])

    return deg_kernel


def _make_prop_kernel(epad, npad, D):
    ec = 512
    nb = ec // BLK
    eblocks = epad // BLK
    bpt = eblocks // NW
    nchunks = bpt // nb
    rpt = npad // NS
    NBUF = 3

    @functools.partial(
        pl.kernel,
        out_type=jax.ShapeDtypeStruct((NC, npad, D), jnp.float32),
        mesh=_sc_mesh(),
        scratch_types=(
            [pltpu.VMEM((nb, BLK), jnp.int32) for _ in range(NBUF)]
            + [pltpu.VMEM((nb, BLK), jnp.int32) for _ in range(NBUF)]
            + [pltpu.VMEM((ec,), jnp.float32) for _ in range(NBUF)]
            + [pltpu.VMEM((ec, D), jnp.float32) for _ in range(NBUF)]
            + [pltpu.VMEM_SHARED((npad, D), jnp.float32)]
            + [pltpu.SemaphoreType.DMA for _ in range(2 * NBUF)]
        ),
        compiler_params=_SC_PARAMS,
    )
    def prop_kernel(src_hbm, dst_hbm, ewf_hbm, g_hbm, out_hbm, *scr):
        sidx = scr[0:NBUF]
        didx = scr[NBUF:2 * NBUF]
        ew = scr[2 * NBUF:3 * NBUF]
        rows = scr[3 * NBUF:4 * NBUF]
        acc_sh = scr[4 * NBUF]
        gsem = scr[4 * NBUF + 1:4 * NBUF + 1 + NBUF]
        ssem = scr[4 * NBUF + 1 + NBUF:]

        core = lax.axis_index("c")
        sub = lax.axis_index("s")
        wid = core * NS + sub
        r0 = sub * rpt

        zero16 = jnp.zeros((L,), jnp.float32)

        @pl.loop(0, min(rpt, ec))
        def _(i):
            for j0 in range(0, D, L):
                rows[0][i, pl.ds(j0, L)] = zero16

        @pl.when(core == 0)
        def _():
            pltpu.sync_copy(g_hbm.at[pl.ds(r0, rpt)], acc_sh.at[pl.ds(r0, rpt)])

        @pl.when(core != 0)
        def _():
            done = 0
            while done < rpt:
                n = min(ec, rpt - done)
                pltpu.sync_copy(rows[0].at[pl.ds(0, n)],
                                acc_sh.at[pl.ds(r0 + done, n)])
                done += n

        plsc.subcore_barrier()

        def load_idx(c):
            b = c % NBUF
            blk0 = wid * bpt + c * nb
            pltpu.sync_copy(src_hbm.at[pl.ds(blk0, nb)], sidx[b])
            pltpu.sync_copy(dst_hbm.at[pl.ds(blk0, nb)], didx[b])
            pltpu.sync_copy(ewf_hbm.at[pl.ds(blk0 * BLK, ec)], ew[b])

        def issue_gathers(c):
            b = c % NBUF
            return [
                pltpu.async_copy(g_hbm.at[sidx[b].at[j]],
                                 rows[b].at[pl.ds(j * BLK, BLK)], gsem[b])
                for j in range(nb)
            ]

        def issue_scatters(c):
            b = c % NBUF
            return [
                pltpu.async_copy(rows[b].at[pl.ds(j * BLK, BLK)],
                                 acc_sh.at[didx[b].at[j]], ssem[b], add=True)
                for j in range(nb)
            ]

        lane_ids = [jnp.full((L,), l, jnp.int32) for l in range(L)]

        def scale(c):
            b = c % NBUF
            rows_b = rows[b]
            ew_b = ew[b]

            @plsc.parallel_loop(0, ec, step=L, unroll=2)
            def _(e0):
                wt16 = ew_b[pl.ds(e0, L)]
                for l in range(L):
                    wb = lax.gather(
                        wt16, lane_ids[l][:, None],
                        lax.GatherDimensionNumbers(
                            offset_dims=(), collapsed_slice_dims=(0,),
                            start_index_map=(0,)),
                        (1,), mode=lax.GatherScatterMode.PROMISE_IN_BOUNDS)
                    for j0 in range(0, D, L):
                        rows_b[e0 + l, pl.ds(j0, L)] = (
                            rows_b[e0 + l, pl.ds(j0, L)] * wb)

        gd = {}
        sd = {}
        load_idx(0)
        gd[0] = issue_gathers(0)
        for c in range(nchunks):
            if c + 1 < nchunks:
                if c - 2 >= 0:
                    for d in sd[c - 2]:
                        d.wait()
                load_idx(c + 1)
                gd[c + 1] = issue_gathers(c + 1)
            for d in gd[c]:
                d.wait()
            scale(c)
            sd[c] = issue_scatters(c)
        for c in range(max(0, nchunks - 3), nchunks):
            for d in sd[c]:
                d.wait()

        plsc.subcore_barrier()
        pltpu.sync_copy(acc_sh.at[pl.ds(r0, rpt)],
                        out_hbm.at[core, pl.ds(r0, rpt)])

    return prop_kernel


def _dis_from_deg(degp):
    deg = 1.0 + degp[0] + degp[1]
    return lax.rsqrt(deg)


def _tc1_body(degp_ref, x_ref, w1_ref, g_ref):
    dis = _dis_from_deg(degp_ref[...])
    h = jnp.dot(x_ref[...], w1_ref[...], preferred_element_type=jnp.float32)
    g_ref[...] = dis[:, None] * h


def _tc2_body(degp_ref, s1_ref, b1_ref, w2_ref, g2_ref):
    dis = _dis_from_deg(degp_ref[...])
    s = s1_ref[...]
    conv1 = dis[:, None] * (s[0] + s[1]) + b1_ref[...]
    o1 = jnp.maximum(conv1, 0.0)
    h2 = jnp.dot(o1, w2_ref[...], preferred_element_type=jnp.float32)
    g2_ref[...] = dis[:, None] * h2


def _tc3_body(C, degp_ref, s2_ref, b2_ref, out_ref):
    dis = _dis_from_deg(degp_ref[...])
    s = s2_ref[...]
    conv2 = dis[:, None] * (s[0] + s[1])[:, :C] + b2_ref[...]
    m = jnp.max(conv2, axis=1, keepdims=True)
    lse = m + jnp.log(jnp.sum(jnp.exp(conv2 - m), axis=1, keepdims=True))
    out_ref[...] = conv2 - lse


def _full2d(shape):
    return pl.BlockSpec(shape, lambda i: (0, 0))


def kernel(x, edge_index, edge_weight, W1, b1, W2, b2):
    N, F = x.shape
    HID = W1.shape[1]
    C = W2.shape[1]
    E = edge_weight.shape[0]

    npad = ((N + BN - 1) // BN) * BN
    epad = ((E + NW * CHUNK - 1) // (NW * CHUNK)) * (NW * CHUNK)
    D2 = ((C + L - 1) // L) * L

    spread = (jnp.arange(epad - E, dtype=jnp.int32) * 37) % N
    src2d = jnp.concatenate([edge_index[0], spread]).reshape(epad // BLK, BLK)
    dst2d = jnp.concatenate([edge_index[1], spread]).reshape(epad // BLK, BLK)
    ewf = jnp.pad(edge_weight, (0, epad - E))
    xp = jnp.pad(x, ((0, npad - N), (0, 0)))
    W2p = jnp.pad(W2, ((0, 0), (0, D2 - C)))
    b1r = b1.reshape(1, HID)
    b2r = b2.reshape(1, C)

    dstf = jnp.concatenate([edge_index[1], spread])
    degp = _make_deg_kernel(epad, npad)(dstf, ewf)

    grid = (npad // BN,)
    degp_spec = pl.BlockSpec((NC, BN), lambda i: (0, i))

    g1 = pl.pallas_call(
        _tc1_body,
        grid=grid,
        in_specs=[degp_spec,
                  pl.BlockSpec((BN, F), lambda i: (i, 0)),
                  _full2d((F, HID))],
        out_specs=pl.BlockSpec((BN, HID), lambda i: (i, 0)),
        out_shape=jax.ShapeDtypeStruct((npad, HID), jnp.float32),
    )(degp, xp, W1)

    s1 = _make_prop_kernel(epad, npad, HID)(src2d, dst2d, ewf, g1)

    g2 = pl.pallas_call(
        _tc2_body,
        grid=grid,
        in_specs=[degp_spec,
                  pl.BlockSpec((NC, BN, HID), lambda i: (0, i, 0)),
                  _full2d((1, HID)),
                  _full2d((HID, D2))],
        out_specs=pl.BlockSpec((BN, D2), lambda i: (i, 0)),
        out_shape=jax.ShapeDtypeStruct((npad, D2), jnp.float32),
    )(degp, s1, b1r, W2p)

    s2 = _make_prop_kernel(epad, npad, D2)(src2d, dst2d, ewf, g2)

    out = pl.pallas_call(
        functools.partial(_tc3_body, C),
        grid=grid,
        in_specs=[degp_spec,
                  pl.BlockSpec((NC, BN, D2), lambda i: (0, i, 0)),
                  _full2d((1, C))],
        out_specs=pl.BlockSpec((BN, C), lambda i: (i, 0)),
        out_shape=jax.ShapeDtypeStruct((npad, C), jnp.float32),
    )(degp, s2, b2r)

    return out[:N]
